# baseline (speedup 1.0000x reference)
import functools

import jax
import jax.numpy as jnp
from jax import lax
from jax.experimental import pallas as pl
from jax.experimental.pallas import tpu as pltpu
from jax.experimental.pallas import tpu_sc as plsc

NC = 2
NS = 16
L = 16
NW = NC * NS
ROWS = 128
COLS = 32768
RPW = ROWS // NW
SEG = 8192
QUARTERS = COLS // SEG
CPS = 16
SEGEL = CPS * L
NSEG = SEG // SEGEL
Y_TC_PAIRS = 384
YPW = (512 - Y_TC_PAIRS) // NW

_BIG_I32 = 2**31 - 1


def _sc_body(x_hbm, z_hbm, y_hbm, outx_hbm, outzv_hbm, outzi_hbm, outy_hbm,
             xb0, xb1, zb0, zb1, yb0, yb1, segx, segz, ox, ozv, ozi, oy,
             semx, semz, semy):
  w = lax.axis_index("s") * NC + lax.axis_index("c")
  row0 = w * RPW
  ypair0 = Y_TC_PAIRS + w * YPW

  iota = lax.iota(jnp.int32, L)
  negv = jnp.full((L,), -jnp.inf, jnp.float32)
  zeroi = jnp.zeros((L,), jnp.int32)

  stages = [(r, q) for r in range(RPW) for q in range(QUARTERS)]
  xzbufs = [(xb0, zb0), (xb1, zb1)]
  ybufs = [yb0, yb1]

  def xz_dma(s):
    r, q = stages[s]
    xb, zb = xzbufs[s % 2]
    cx = pltpu.async_copy(x_hbm.at[row0 + r, pl.ds(q * SEG, SEG)], xb, semx)
    cz = pltpu.async_copy(z_hbm.at[row0 + r, pl.ds(q * SEG, SEG)], zb, semz)
    return cx, cz

  def y_dma(j):
    return pltpu.async_copy(y_hbm.at[ypair0 + j], ybufs[j % 2], semy)

  def kth_lane_max(g, k):
    for _ in range(k - 1):
      t = jnp.max(g)
      eq = g == t
      sel = eq & (iota == plsc.all_reduce_ffs(eq))
      g = jnp.where(sel, negv, g)
    return jnp.max(g)

  cur = xz_dma(0)
  ycur = y_dma(0)
  carry = None
  for s, (r, q) in enumerate(stages):
    nxt = xz_dma(s + 1) if s + 1 < len(stages) else None
    cx, cz = cur
    cx.wait()
    cz.wait()
    cur = nxt
    xb, zb = xzbufs[s % 2]
    if q == 0:
      carry = (negv, negv, negv, negv, negv, negv, negv, zeroi, zeroi, zeroi)

    def p1(sg, c, xb=xb, zb=zb):
      gx, gz = c
      mx = negv
      mz = negv
      for u in range(CPS):
        off = sg * SEGEL + u * L
        mx = jnp.maximum(mx, xb[pl.ds(off, L)])
        mz = jnp.maximum(mz, zb[pl.ds(off, L)])
      segx[pl.ds(sg * L, L)] = mx
      segz[pl.ds(sg * L, L)] = mz
      return (jnp.maximum(gx, mx), jnp.maximum(gz, mz))

    gx, gz = lax.fori_loop(0, NSEG, p1, (negv, negv))
    tx = kth_lane_max(gx, 4)
    tz = kth_lane_max(gz, 3)

    def p2(sg, c, xb=xb, zb=zb, q=q, tx=tx, tz=tz):
      x0, x1, x2, x3, v0, v1, v2, i0, i1, i2 = c

      def xtrue(xc):
        x0, x1, x2, x3 = xc
        for u in range(CPS):
          off = sg * SEGEL + u * L
          xv = xb[pl.ds(off, L)]
          cm = xv > x0
          t = jnp.where(cm, x0, xv); x0 = jnp.where(cm, xv, x0); xv = t
          cm = xv > x1
          t = jnp.where(cm, x1, xv); x1 = jnp.where(cm, xv, x1); xv = t
          cm = xv > x2
          t = jnp.where(cm, x2, xv); x2 = jnp.where(cm, xv, x2); xv = t
          x3 = jnp.maximum(x3, xv)
        return (x0, x1, x2, x3)

      hitx = jnp.max(segx[pl.ds(sg * L, L)]) >= tx
      x0, x1, x2, x3 = lax.cond(hitx, xtrue, lambda xc: xc, (x0, x1, x2, x3))

      def ztrue(zc):
        v0, v1, v2, i0, i1, i2 = zc
        for u in range(CPS):
          off = sg * SEGEL + u * L
          zv = zb[pl.ds(off, L)]
          zx = iota + (q * SEG + off)
          cm = zv > v0
          tv = jnp.where(cm, v0, zv); ti = jnp.where(cm, i0, zx)
          v0 = jnp.where(cm, zv, v0); i0 = jnp.where(cm, zx, i0)
          zv, zx = tv, ti
          cm = zv > v1
          tv = jnp.where(cm, v1, zv); ti = jnp.where(cm, i1, zx)
          v1 = jnp.where(cm, zv, v1); i1 = jnp.where(cm, zx, i1)
          zv, zx = tv, ti
          cm = zv > v2
          v2 = jnp.where(cm, zv, v2); i2 = jnp.where(cm, zx, i2)
        return (v0, v1, v2, i0, i1, i2)

      hitz = jnp.max(segz[pl.ds(sg * L, L)]) >= tz
      v0, v1, v2, i0, i1, i2 = lax.cond(
          hitz, ztrue, lambda zc: zc, (v0, v1, v2, i0, i1, i2))
      return (x0, x1, x2, x3, v0, v1, v2, i0, i1, i2)

    carry = lax.fori_loop(0, NSEG, p2, carry)

    if q == QUARTERS - 1:
      x0, x1, x2, x3, v0, v1, v2, i0, i1, i2 = carry
      resx = negv
      for j in range(4):
        mx = jnp.max(x0)
        resx = jnp.where(iota == j, mx, resx)
        eq = x0 == mx
        sel = eq & (iota == plsc.all_reduce_ffs(eq))
        x0 = jnp.where(sel, x1, x0)
        x1 = jnp.where(sel, x2, x1)
        x2 = jnp.where(sel, x3, x2)
        x3 = jnp.where(sel, negv, x3)
      ox[r] = resx
      reszv = negv
      reszi = zeroi
      for j in range(3):
        mz = jnp.max(v0)
        eq = v0 == mz
        mi = jnp.min(jnp.where(eq, i0, _BIG_I32))
        reszv = jnp.where(iota == j, mz, reszv)
        reszi = jnp.where(iota == j, mi, reszi)
        sel = eq & (i0 == mi)
        v0 = jnp.where(sel, v1, v0); i0 = jnp.where(sel, i1, i0)
        v1 = jnp.where(sel, v2, v1); i1 = jnp.where(sel, i2, i1)
        v2 = jnp.where(sel, negv, v2); i2 = jnp.where(sel, zeroi, i2)
      ozv[r] = reszv
      ozi[r] = reszi

  for j in range(YPW):
    ynxt = y_dma(j + 1) if j + 1 < YPW else None
    ycur.wait()
    ycur = ynxt
    yb = ybufs[j % 2]
    res = negv
    for c in range(8):
      def yrow(k, acc, yb=yb, c=c):
        for u in range(8):
          acc = jnp.minimum(acc, yb[c, pl.ds((k * 8 + u) * L, L)])
        return acc
      acc = lax.fori_loop(0, 4096 // (8 * L), yrow, jnp.full((L,), jnp.inf,
                                                            jnp.float32))
      res = jnp.where(iota == c, jnp.min(acc), res)
    oy[j] = res

  pltpu.sync_copy(ox, outx_hbm.at[pl.ds(row0, RPW)])
  pltpu.sync_copy(ozv, outzv_hbm.at[pl.ds(row0, RPW)])
  pltpu.sync_copy(ozi, outzi_hbm.at[pl.ds(row0, RPW)])
  pltpu.sync_copy(oy, outy_hbm.at[pl.ds(w * YPW, YPW)])


_sc_topk = functools.partial(
    pl.kernel,
    mesh=plsc.VectorSubcoreMesh(core_axis_name="c", subcore_axis_name="s"),
    compiler_params=pltpu.CompilerParams(needs_layout_passes=False),
    out_type=[
        jax.ShapeDtypeStruct((ROWS, 16), jnp.float32),
        jax.ShapeDtypeStruct((ROWS, 16), jnp.float32),
        jax.ShapeDtypeStruct((ROWS, 16), jnp.int32),
        jax.ShapeDtypeStruct((512 - Y_TC_PAIRS, 16), jnp.float32),
    ],
    scratch_types=[
        pltpu.VMEM((SEG,), jnp.float32),
        pltpu.VMEM((SEG,), jnp.float32),
        pltpu.VMEM((SEG,), jnp.float32),
        pltpu.VMEM((SEG,), jnp.float32),
        pltpu.VMEM((8, 4096), jnp.float32),
        pltpu.VMEM((8, 4096), jnp.float32),
        pltpu.VMEM((NSEG * L,), jnp.float32),
        pltpu.VMEM((NSEG * L,), jnp.float32),
        pltpu.VMEM((RPW, 16), jnp.float32),
        pltpu.VMEM((RPW, 16), jnp.float32),
        pltpu.VMEM((RPW, 16), jnp.int32),
        pltpu.VMEM((YPW, 16), jnp.float32),
        pltpu.SemaphoreType.DMA,
        pltpu.SemaphoreType.DMA,
        pltpu.SemaphoreType.DMA,
    ],
)(_sc_body)


YB = 16


def _ymin_body(y_ref, o_ref):
  o_ref[...] = jnp.min(y_ref[...], axis=2)


_ymin = pl.pallas_call(
    _ymin_body,
    grid=(Y_TC_PAIRS // YB,),
    in_specs=[pl.BlockSpec((YB, 8, 4096), lambda i: (i, 0, 0))],
    out_specs=pl.BlockSpec((YB, 8), lambda i: (i, 0)),
    out_shape=jax.ShapeDtypeStruct((Y_TC_PAIRS, 8), jnp.float32),
)


def kernel(x, y, z):
  yt = jnp.moveaxis(y, 2, 3).reshape(512, 8, 4096)
  xo, zvo, zio, ysc = _sc_topk(x, z, yt)
  ytc = _ymin(yt)
  yv = jnp.concatenate([ytc, ysc[:, :8]], axis=0).reshape(32, 16, 1, 8)
  return (xo[:, :4], yv, zvo[:, :3], zio[:, :3])


# speedup vs baseline: 1.2475x; 1.2475x over previous
import functools

import jax
import jax.numpy as jnp
from jax import lax
from jax.experimental import pallas as pl
from jax.experimental.pallas import tpu as pltpu
from jax.experimental.pallas import tpu_sc as plsc

NC = 2
NS = 16
L = 16
NW = NC * NS
ROWS = 128
COLS = 32768
RPW = ROWS // NW
SEG = 8192
QUARTERS = COLS // SEG
CPS = 16
SEGEL = CPS * L
NSEG = SEG // SEGEL

_BIG_I32 = 2**31 - 1


def _sc_body(x_hbm, z_hbm, outx_hbm, outzv_hbm, outzi_hbm,
             xb0, xb1, zb0, zb1, segx, segz, ox, ozv, ozi,
             semx, semz):
  w = lax.axis_index("s") * NC + lax.axis_index("c")
  row0 = w * RPW

  iota = lax.iota(jnp.int32, L)
  negv = jnp.full((L,), -jnp.inf, jnp.float32)
  zeroi = jnp.zeros((L,), jnp.int32)

  stages = [(r, q) for r in range(RPW) for q in range(QUARTERS)]
  xzbufs = [(xb0, zb0), (xb1, zb1)]

  def xz_dma(s):
    r, q = stages[s]
    xb, zb = xzbufs[s % 2]
    cx = pltpu.async_copy(x_hbm.at[row0 + r, pl.ds(q * SEG, SEG)], xb, semx)
    cz = pltpu.async_copy(z_hbm.at[row0 + r, pl.ds(q * SEG, SEG)], zb, semz)
    return cx, cz

  def pop_one(g, rest):
    t = jnp.max(g)
    eq = g == t
    sel = eq & (iota == plsc.all_reduce_ffs(eq))
    cols = (g,) + rest
    out = []
    for i in range(len(cols)):
      nxt = cols[i + 1] if i + 1 < len(cols) else negv
      out.append(jnp.where(sel, nxt, cols[i]))
    return out, t

  def kth_of_cols(cols, k):
    cur = list(cols)
    t = None
    for _ in range(k):
      cur, t = pop_one(cur[0], tuple(cur[1:]))
    return t

  cur = xz_dma(0)
  carry = None
  for s, (r, q) in enumerate(stages):
    nxt = xz_dma(s + 1) if s + 1 < len(stages) else None
    cx, cz = cur
    cx.wait()
    cz.wait()
    cur = nxt
    xb, zb = xzbufs[s % 2]
    if q == 0:
      carry = (negv, negv, negv, negv, negv, negv, negv, zeroi, zeroi, zeroi)
    x0, x1, x2, x3, v0, v1, v2, i0, i1, i2 = carry

    def p1(sg, c, xb=xb, zb=zb):
      gx, gz = c
      mx = negv
      mz = negv
      for u in range(CPS):
        off = sg * SEGEL + u * L
        mx = jnp.maximum(mx, xb[pl.ds(off, L)])
        mz = jnp.maximum(mz, zb[pl.ds(off, L)])
      segx[pl.ds(sg * L, L)] = mx
      segz[pl.ds(sg * L, L)] = mz
      return (jnp.maximum(gx, mx), jnp.maximum(gz, mz))

    gx, gz = lax.fori_loop(0, NSEG, p1, (negv, negv))
    tx = jnp.maximum(kth_of_cols((gx,), 4), kth_of_cols((x0, x1, x2, x3), 4))
    tz = jnp.maximum(kth_of_cols((gz,), 3), kth_of_cols((v0, v1, v2), 3))

    def p2(sg, c, xb=xb, zb=zb, q=q, tx=tx, tz=tz):
      def hit_any(c):
        x0, x1, x2, x3, v0, v1, v2, i0, i1, i2 = c
        smx = segx[pl.ds(sg * L, L)]
        smz = segz[pl.ds(sg * L, L)]

        def xtrue(xc):
          x0, x1, x2, x3 = xc
          for u in range(CPS):
            off = sg * SEGEL + u * L
            xv = xb[pl.ds(off, L)]
            cm = xv > x0
            t = jnp.where(cm, x0, xv); x0 = jnp.where(cm, xv, x0); xv = t
            cm = xv > x1
            t = jnp.where(cm, x1, xv); x1 = jnp.where(cm, xv, x1); xv = t
            cm = xv > x2
            t = jnp.where(cm, x2, xv); x2 = jnp.where(cm, xv, x2); xv = t
            x3 = jnp.maximum(x3, xv)
          return (x0, x1, x2, x3)

        x0, x1, x2, x3 = lax.cond(
            jnp.max(smx) >= tx, xtrue, lambda xc: xc, (x0, x1, x2, x3))

        def ztrue(zc):
          v0, v1, v2, i0, i1, i2 = zc
          for u in range(CPS):
            off = sg * SEGEL + u * L
            zv = zb[pl.ds(off, L)]
            zx = iota + (q * SEG + off)
            cm = zv > v0
            tv = jnp.where(cm, v0, zv); ti = jnp.where(cm, i0, zx)
            v0 = jnp.where(cm, zv, v0); i0 = jnp.where(cm, zx, i0)
            zv, zx = tv, ti
            cm = zv > v1
            tv = jnp.where(cm, v1, zv); ti = jnp.where(cm, i1, zx)
            v1 = jnp.where(cm, zv, v1); i1 = jnp.where(cm, zx, i1)
            zv, zx = tv, ti
            cm = zv > v2
            v2 = jnp.where(cm, zv, v2); i2 = jnp.where(cm, zx, i2)
          return (v0, v1, v2, i0, i1, i2)

        v0, v1, v2, i0, i1, i2 = lax.cond(
            jnp.max(smz) >= tz, ztrue, lambda zc: zc,
            (v0, v1, v2, i0, i1, i2))
        return (x0, x1, x2, x3, v0, v1, v2, i0, i1, i2)

      smx = segx[pl.ds(sg * L, L)]
      smz = segz[pl.ds(sg * L, L)]
      m = jnp.maximum(smx - tx, smz - tz)
      return lax.cond(jnp.max(m) >= 0.0, hit_any, lambda c: c, c)

    carry = lax.fori_loop(0, NSEG, p2, carry)

    if q == QUARTERS - 1:
      x0, x1, x2, x3, v0, v1, v2, i0, i1, i2 = carry
      resx = negv
      cols = [x0, x1, x2, x3]
      for j in range(4):
        cols, mx = pop_one(cols[0], tuple(cols[1:]))
        resx = jnp.where(iota == j, mx, resx)
      ox[r] = resx
      reszv = negv
      reszi = zeroi
      for j in range(3):
        mz = jnp.max(v0)
        eq = v0 == mz
        mi = jnp.min(jnp.where(eq, i0, _BIG_I32))
        reszv = jnp.where(iota == j, mz, reszv)
        reszi = jnp.where(iota == j, mi, reszi)
        sel = eq & (i0 == mi)
        v0 = jnp.where(sel, v1, v0); i0 = jnp.where(sel, i1, i0)
        v1 = jnp.where(sel, v2, v1); i1 = jnp.where(sel, i2, i1)
        v2 = jnp.where(sel, negv, v2); i2 = jnp.where(sel, zeroi, i2)
      ozv[r] = reszv
      ozi[r] = reszi

  pltpu.sync_copy(ox, outx_hbm.at[pl.ds(row0, RPW)])
  pltpu.sync_copy(ozv, outzv_hbm.at[pl.ds(row0, RPW)])
  pltpu.sync_copy(ozi, outzi_hbm.at[pl.ds(row0, RPW)])


_sc_topk = functools.partial(
    pl.kernel,
    mesh=plsc.VectorSubcoreMesh(core_axis_name="c", subcore_axis_name="s"),
    compiler_params=pltpu.CompilerParams(needs_layout_passes=False),
    out_type=[
        jax.ShapeDtypeStruct((ROWS, 16), jnp.float32),
        jax.ShapeDtypeStruct((ROWS, 16), jnp.float32),
        jax.ShapeDtypeStruct((ROWS, 16), jnp.int32),
    ],
    scratch_types=[
        pltpu.VMEM((SEG,), jnp.float32),
        pltpu.VMEM((SEG,), jnp.float32),
        pltpu.VMEM((SEG,), jnp.float32),
        pltpu.VMEM((SEG,), jnp.float32),
        pltpu.VMEM((NSEG * L,), jnp.float32),
        pltpu.VMEM((NSEG * L,), jnp.float32),
        pltpu.VMEM((RPW, 16), jnp.float32),
        pltpu.VMEM((RPW, 16), jnp.float32),
        pltpu.VMEM((RPW, 16), jnp.int32),
        pltpu.SemaphoreType.DMA,
        pltpu.SemaphoreType.DMA,
    ],
)(_sc_body)


YB = 16


def _ymin_body(y_ref, o_ref):
  o_ref[...] = jnp.min(y_ref[...], axis=2)


_ymin = pl.pallas_call(
    _ymin_body,
    grid=(512 // YB,),
    in_specs=[pl.BlockSpec((YB, 8, 4096), lambda i: (i, 0, 0))],
    out_specs=pl.BlockSpec((YB, 8), lambda i: (i, 0)),
    out_shape=jax.ShapeDtypeStruct((512, 8), jnp.float32),
)


def kernel(x, y, z):
  yt = jnp.moveaxis(y, 2, 3).reshape(512, 8, 4096)
  xo, zvo, zio = _sc_topk(x, z)
  ym = _ymin(yt)
  return (xo[:, :4], ym.reshape(32, 16, 1, 8), zvo[:, :3], zio[:, :3])


# speedup vs baseline: 1.4264x; 1.1433x over previous
import functools

import jax
import jax.numpy as jnp
from jax import lax
from jax.experimental import pallas as pl
from jax.experimental.pallas import tpu as pltpu
from jax.experimental.pallas import tpu_sc as plsc

NC = 2
NS = 16
L = 16
NW = NC * NS
ROWS = 128
COLS = 32768
RPW = ROWS // NW
SEG = 8192
QUARTERS = COLS // SEG
CPS = 16
SEGEL = CPS * L
NSEG = SEG // SEGEL

_BIG_I32 = 2**31 - 1


def _sc_body(x_hbm, z_hbm, outx_hbm, outzv_hbm, outzi_hbm,
             xb0, xb1, zb0, zb1, segx, segz, ox, ozv, ozi,
             semx, semz):
  w = lax.axis_index("s") * NC + lax.axis_index("c")
  row0 = w * RPW

  iota = lax.iota(jnp.int32, L)
  negv = jnp.full((L,), -jnp.inf, jnp.float32)
  zeroi = jnp.zeros((L,), jnp.int32)
  xzbufs = [(xb0, zb0), (xb1, zb1)]

  def xz_dma(row, q):
    xb, zb = xzbufs[q % 2]
    cx = pltpu.async_copy(x_hbm.at[row, pl.ds(q * SEG, SEG)], xb, semx)
    cz = pltpu.async_copy(z_hbm.at[row, pl.ds(q * SEG, SEG)], zb, semz)
    return cx, cz

  def xz_wait(q):
    xb, zb = xzbufs[q % 2]
    pltpu.make_async_copy(x_hbm.at[0, pl.ds(0, SEG)], xb, semx).wait()
    pltpu.make_async_copy(z_hbm.at[0, pl.ds(0, SEG)], zb, semz).wait()

  def pop_one(cols):
    g = cols[0]
    t = jnp.max(g)
    eq = g == t
    sel = eq & (iota == plsc.all_reduce_ffs(eq))
    out = []
    for i in range(len(cols)):
      nxt = cols[i + 1] if i + 1 < len(cols) else negv
      out.append(jnp.where(sel, nxt, cols[i]))
    return out, t

  def kth_of_cols(cols, k):
    cur = list(cols)
    t = None
    for _ in range(k):
      cur, t = pop_one(cur)
    return t

  xz_dma(row0, 0)

  def rbody(r, _):
    row = row0 + r
    carry = (negv, negv, negv, negv, negv, negv, negv, zeroi, zeroi, zeroi)
    for q in range(QUARTERS):
      xz_wait(q)
      if q + 1 < QUARTERS:
        xz_dma(row, q + 1)
      else:
        xz_dma(jnp.minimum(row + 1, ROWS - 1), 0)
      xb, zb = xzbufs[q % 2]
      x0, x1, x2, x3, v0, v1, v2, i0, i1, i2 = carry

      def p1(sg, c, xb=xb, zb=zb):
        gx, gz = c
        mx = negv
        mz = negv
        for u in range(CPS):
          off = sg * SEGEL + u * L
          mx = jnp.maximum(mx, xb[pl.ds(off, L)])
          mz = jnp.maximum(mz, zb[pl.ds(off, L)])
        segx[pl.ds(sg * L, L)] = mx
        segz[pl.ds(sg * L, L)] = mz
        return (jnp.maximum(gx, mx), jnp.maximum(gz, mz))

      gx, gz = lax.fori_loop(0, NSEG, p1, (negv, negv))
      tx = jnp.maximum(kth_of_cols((gx,), 4),
                       kth_of_cols((x0, x1, x2, x3), 4))
      tz = jnp.maximum(kth_of_cols((gz,), 3), kth_of_cols((v0, v1, v2), 3))

      def p2(sgp, c, xb=xb, zb=zb, q=q, tx=tx, tz=tz):
        def hit_pair(c):
          def one_seg(sg, c):
            x0, x1, x2, x3, v0, v1, v2, i0, i1, i2 = c

            def xtrue(xc):
              def xu(u, xc):
                x0, x1, x2, x3 = xc
                off = sg * SEGEL + u * L
                xv = xb[pl.ds(off, L)]
                cm = xv > x0
                t = jnp.where(cm, x0, xv); x0 = jnp.where(cm, xv, x0); xv = t
                cm = xv > x1
                t = jnp.where(cm, x1, xv); x1 = jnp.where(cm, xv, x1); xv = t
                cm = xv > x2
                t = jnp.where(cm, x2, xv); x2 = jnp.where(cm, xv, x2); xv = t
                x3 = jnp.maximum(x3, xv)
                return (x0, x1, x2, x3)
              return lax.fori_loop(0, CPS, xu, xc, unroll=4)

            x0, x1, x2, x3 = lax.cond(
                jnp.max(segx[pl.ds(sg * L, L)]) >= tx,
                xtrue, lambda xc: xc, (x0, x1, x2, x3))

            def ztrue(zc):
              def zu(u, zc):
                v0, v1, v2, i0, i1, i2 = zc
                off = sg * SEGEL + u * L
                zv = zb[pl.ds(off, L)]
                zx = iota + (q * SEG + off)
                cm = zv > v0
                tv = jnp.where(cm, v0, zv); ti = jnp.where(cm, i0, zx)
                v0 = jnp.where(cm, zv, v0); i0 = jnp.where(cm, zx, i0)
                zv, zx = tv, ti
                cm = zv > v1
                tv = jnp.where(cm, v1, zv); ti = jnp.where(cm, i1, zx)
                v1 = jnp.where(cm, zv, v1); i1 = jnp.where(cm, zx, i1)
                zv, zx = tv, ti
                cm = zv > v2
                v2 = jnp.where(cm, zv, v2); i2 = jnp.where(cm, zx, i2)
                return (v0, v1, v2, i0, i1, i2)
              return lax.fori_loop(0, CPS, zu, zc, unroll=4)

            v0, v1, v2, i0, i1, i2 = lax.cond(
                jnp.max(segz[pl.ds(sg * L, L)]) >= tz,
                ztrue, lambda zc: zc, (v0, v1, v2, i0, i1, i2))
            return (x0, x1, x2, x3, v0, v1, v2, i0, i1, i2)

          c = one_seg(2 * sgp, c)
          return one_seg(2 * sgp + 1, c)

        smx = jnp.maximum(segx[pl.ds(2 * sgp * L, L)],
                          segx[pl.ds((2 * sgp + 1) * L, L)])
        smz = jnp.maximum(segz[pl.ds(2 * sgp * L, L)],
                          segz[pl.ds((2 * sgp + 1) * L, L)])
        m = jnp.maximum(smx - tx, smz - tz)
        return lax.cond(jnp.max(m) >= 0.0, hit_pair, lambda c: c, c)

      carry = lax.fori_loop(0, NSEG // 2, p2, carry)

    x0, x1, x2, x3, v0, v1, v2, i0, i1, i2 = carry
    resx = negv
    cols = [x0, x1, x2, x3]
    for j in range(4):
      cols, mx = pop_one(cols)
      resx = jnp.where(iota == j, mx, resx)
    ox[pl.ds(r * L, L)] = resx
    reszv = negv
    reszi = zeroi
    for j in range(3):
      mz = jnp.max(v0)
      eq = v0 == mz
      mi = jnp.min(jnp.where(eq, i0, _BIG_I32))
      reszv = jnp.where(iota == j, mz, reszv)
      reszi = jnp.where(iota == j, mi, reszi)
      sel = eq & (i0 == mi)
      v0 = jnp.where(sel, v1, v0); i0 = jnp.where(sel, i1, i0)
      v1 = jnp.where(sel, v2, v1); i1 = jnp.where(sel, i2, i1)
      v2 = jnp.where(sel, negv, v2); i2 = jnp.where(sel, zeroi, i2)
    ozv[pl.ds(r * L, L)] = reszv
    ozi[pl.ds(r * L, L)] = reszi
    return 0

  lax.fori_loop(0, RPW, rbody, 0)
  xz_wait(0)

  pltpu.sync_copy(ox, outx_hbm.at[pl.ds(row0 * L, RPW * L)])
  pltpu.sync_copy(ozv, outzv_hbm.at[pl.ds(row0 * L, RPW * L)])
  pltpu.sync_copy(ozi, outzi_hbm.at[pl.ds(row0 * L, RPW * L)])


_sc_topk = functools.partial(
    pl.kernel,
    mesh=plsc.VectorSubcoreMesh(core_axis_name="c", subcore_axis_name="s"),
    compiler_params=pltpu.CompilerParams(needs_layout_passes=False),
    out_type=[
        jax.ShapeDtypeStruct((ROWS * 16,), jnp.float32),
        jax.ShapeDtypeStruct((ROWS * 16,), jnp.float32),
        jax.ShapeDtypeStruct((ROWS * 16,), jnp.int32),
    ],
    scratch_types=[
        pltpu.VMEM((SEG,), jnp.float32),
        pltpu.VMEM((SEG,), jnp.float32),
        pltpu.VMEM((SEG,), jnp.float32),
        pltpu.VMEM((SEG,), jnp.float32),
        pltpu.VMEM((NSEG * L,), jnp.float32),
        pltpu.VMEM((NSEG * L,), jnp.float32),
        pltpu.VMEM((RPW * L,), jnp.float32),
        pltpu.VMEM((RPW * L,), jnp.float32),
        pltpu.VMEM((RPW * L,), jnp.int32),
        pltpu.SemaphoreType.DMA,
        pltpu.SemaphoreType.DMA,
    ],
)(_sc_body)


YB = 16


def _ymin_body(y_ref, o_ref):
  o_ref[...] = jnp.min(y_ref[...], axis=2)


_ymin = pl.pallas_call(
    _ymin_body,
    grid=(512 // YB,),
    in_specs=[pl.BlockSpec((YB, 8, 4096), lambda i: (i, 0, 0))],
    out_specs=pl.BlockSpec((YB, 8), lambda i: (i, 0)),
    out_shape=jax.ShapeDtypeStruct((512, 8), jnp.float32),
)


def kernel(x, y, z):
  yt = jnp.moveaxis(y, 2, 3).reshape(512, 8, 4096)
  xo, zvo, zio = _sc_topk(x, z)
  ym = _ymin(yt)
  return (xo.reshape(ROWS, 16)[:, :4], ym.reshape(32, 16, 1, 8),
          zvo.reshape(ROWS, 16)[:, :3], zio.reshape(ROWS, 16)[:, :3])
